# baseline (speedup 1.0000x reference)
import functools

import jax
import jax.numpy as jnp
from jax import lax
from jax.experimental import pallas as pl
from jax.experimental.pallas import tpu as pltpu
from jax.experimental.pallas import tpu_sc as plsc

P = 128
B = 128
H = 256
PB = P * B
ALPHA = 0.5
UNIF_CONST = (1.0 - ALPHA) / P

import numpy as np

_TINY = np.float32(1.1754943508222875e-38)
_ONE_MINUS_TINY = np.float32(1.0)


def _threefry2x32_zero(cnt):
    k0 = jnp.uint32(0)
    k1 = jnp.uint32(42)
    ks = [k0, k1, k0 ^ k1 ^ jnp.uint32(0x1BD11BDA)]
    rot_groups = ((13, 15, 26, 6), (17, 29, 16, 24))

    x1 = cnt + ks[1]
    x0 = x1
    first = True
    for i in range(5):
        for r in rot_groups[i % 2]:
            if first:
                first = False
            else:
                x0 = x0 + x1
            x1 = (x1 << jnp.uint32(r)) | (x1 >> jnp.uint32(32 - r))
            x1 = x1 ^ x0
        x0 = x0 + ks[(i + 1) % 3]
        x1 = x1 + ks[(i + 2) % 3] + jnp.uint32(i + 1)
    return x0, x1


def _sample_body(prob_pb_ref, idx_ref, out_ref, pre_ref):
    prob_pb = prob_pb_ref[...]

    l_pb = jnp.log(ALPHA * jnp.exp(prob_pb) + UNIF_CONST)
    d = prob_pb - l_pb

    j_i = lax.broadcasted_iota(jnp.uint32, (P, B), 0)
    b_i = lax.broadcasted_iota(jnp.uint32, (P, B), 1)
    cnt0 = b_i * jnp.uint32(P) + j_i
    jcol = lax.broadcasted_iota(jnp.int32, (P, B), 0)
    bline = lax.broadcasted_iota(jnp.int32, (1, B), 1)

    def one_row(s):
        cnt = cnt0 + (s * (B * P)).astype(jnp.uint32)
        o0, o1 = _threefry2x32_zero(cnt)
        bits = o0 ^ o1

        float_bits = (bits >> jnp.uint32(9)) | jnp.uint32(0x3F800000)
        floats = lax.bitcast_convert_type(float_bits, jnp.float32)
        floats = floats - jnp.float32(1.0)
        u = jnp.maximum(_TINY, floats * _ONE_MINUS_TINY + _TINY)
        g = -jnp.log(-jnp.log(u))

        vals = g + l_pb
        m = jnp.max(vals, axis=0, keepdims=True)
        cand = jnp.where(vals == m, jcol, jnp.int32(P))
        idx_row = jnp.min(cand, axis=0, keepdims=True)

        idx_ref[pl.ds(s, 1), :] = bline + idx_row * B

        mask = idx_row == jcol
        pre_ref[pl.ds(s, 1), :] = jnp.sum(
            jnp.where(mask, d, jnp.float32(0.0)), axis=0, keepdims=True
        )

    def step(k, _):
        one_row(2 * k)
        one_row(2 * k + 1)
        return 0

    lax.fori_loop(0, P // 2, step, 0)

    pre = pre_ref[...]
    m2 = jnp.max(pre, axis=0, keepdims=True)
    lse = m2 + jnp.log(jnp.sum(jnp.exp(pre - m2), axis=0, keepdims=True))
    out_ref[...] = pre - lse


_sample_call = pl.pallas_call(
    _sample_body,
    out_shape=[
        jax.ShapeDtypeStruct((P, B), jnp.int32),
        jax.ShapeDtypeStruct((P, B), jnp.float32),
    ],
    scratch_shapes=[pltpu.VMEM((P, B), jnp.float32)],
)

NC = 2
NS = 16
NW = NC * NS
ROWS_W = PB // NW
CH = 128
NCHUNK = ROWS_W // CH
IDX_ROWS_W = ROWS_W // B

@functools.lru_cache(maxsize=None)
def _make_gather_call():
    mesh = plsc.VectorSubcoreMesh(
        core_axis_name="c", subcore_axis_name="s", num_cores=NC, num_subcores=NS
    )

    @functools.partial(
        pl.kernel,
        mesh=mesh,
        out_type=jax.ShapeDtypeStruct((PB, H), jnp.float32),
        scratch_types=[
            pltpu.VMEM((NCHUNK, CH), jnp.int32),
            pltpu.VMEM((2, CH, H), jnp.float32),
            pltpu.SemaphoreType.DMA,
            pltpu.SemaphoreType.DMA,
        ],
    )
    def gather_call(idx_hbm, parts_hbm, out_hbm, idx_v, buf_v, sem0, sem1):
        wid = lax.axis_index("s") * NC + lax.axis_index("c")
        pltpu.sync_copy(idx_hbm.at[pl.ds(wid * IDX_ROWS_W, IDX_ROWS_W)], idx_v)
        sems = (sem0, sem1)
        copies = [None, None]
        copies[0] = pltpu.async_copy(parts_hbm.at[idx_v.at[0]], buf_v.at[0], sem0)
        for c in range(NCHUNK):
            nxt = c + 1
            if nxt < NCHUNK:
                copies[nxt % 2] = pltpu.async_copy(
                    parts_hbm.at[idx_v.at[nxt]], buf_v.at[nxt % 2], sems[nxt % 2]
                )
            copies[c % 2].wait()
            pltpu.sync_copy(
                buf_v.at[c % 2], out_hbm.at[pl.ds(wid * ROWS_W + c * CH, CH)]
            )

    return gather_call


def kernel(particles, prob):
    prob_pb = prob.reshape(P, B)
    flat_idx, prob_new = _sample_call(prob_pb)
    particles_new = _make_gather_call()(flat_idx, particles)
    return particles_new, prob_new.reshape(P, B, 1)


# speedup vs baseline: 1.0048x; 1.0048x over previous
import functools

import jax
import jax.numpy as jnp
import numpy as np
from jax import lax
from jax.experimental import pallas as pl
from jax.experimental.pallas import tpu as pltpu
from jax.experimental.pallas import tpu_sc as plsc

P = 128
B = 128
H = 256
PB = P * B
ALPHA = 0.5
UNIF_CONST = (1.0 - ALPHA) / P

HALF = P // 2
UNROLL = 4

_TINY = np.float32(1.1754943508222875e-38)
_ONE_MINUS_TINY = np.float32(1.0)


def _threefry2x32_zero(cnt):
    k0 = jnp.uint32(0)
    k1 = jnp.uint32(42)
    ks = [k0, k1, k0 ^ k1 ^ jnp.uint32(0x1BD11BDA)]
    rot_groups = ((13, 15, 26, 6), (17, 29, 16, 24))

    x1 = cnt + ks[1]
    x0 = x1
    first = True
    for i in range(5):
        for r in rot_groups[i % 2]:
            if first:
                first = False
            else:
                x0 = x0 + x1
            x1 = (x1 << jnp.uint32(r)) | (x1 >> jnp.uint32(32 - r))
            x1 = x1 ^ x0
        x0 = x0 + ks[(i + 1) % 3]
        x1 = x1 + ks[(i + 2) % 3] + jnp.uint32(i + 1)
    return x0, x1


def _sample_half_rows(prob_pb, s_base, idx_ref, pre_store):
    l_pb = jnp.log(ALPHA * jnp.exp(prob_pb) + UNIF_CONST)
    d = prob_pb - l_pb

    j_i = lax.broadcasted_iota(jnp.uint32, (P, B), 0)
    b_i = lax.broadcasted_iota(jnp.uint32, (P, B), 1)
    cnt0 = b_i * jnp.uint32(P) + j_i
    jcol = lax.broadcasted_iota(jnp.int32, (P, B), 0)
    bline = lax.broadcasted_iota(jnp.int32, (1, B), 1)

    def one_row(s_local):
        s = s_local + s_base
        cnt = cnt0 + (s * (B * P)).astype(jnp.uint32)
        o0, o1 = _threefry2x32_zero(cnt)
        bits = o0 ^ o1

        float_bits = (bits >> jnp.uint32(9)) | jnp.uint32(0x3F800000)
        floats = lax.bitcast_convert_type(float_bits, jnp.float32)
        floats = floats - jnp.float32(1.0)
        u = jnp.maximum(_TINY, floats * _ONE_MINUS_TINY + _TINY)
        g = -jnp.log(-jnp.log(u))

        vals = g + l_pb
        m = jnp.max(vals, axis=0, keepdims=True)
        cand = jnp.where(vals == m, jcol, jnp.int32(P))
        idx_row = jnp.min(cand, axis=0, keepdims=True)

        idx_ref[pl.ds(s_local, 1), :] = bline + idx_row * B

        mask = idx_row == jcol
        pre_store(
            s_local,
            jnp.sum(jnp.where(mask, d, jnp.float32(0.0)), axis=0, keepdims=True),
        )

    def step(k, _):
        for r in range(UNROLL):
            one_row(UNROLL * k + r)
        return 0

    lax.fori_loop(0, HALF // UNROLL, step, 0)


def _sample_a_body(prob_pb_ref, idx_ref, pre_ref):
    prob_pb = prob_pb_ref[...]

    def pre_store(s_local, row):
        pre_ref[pl.ds(s_local, 1), :] = row

    _sample_half_rows(prob_pb, 0, idx_ref, pre_store)


def _sample_b_body(prob_pb_ref, pre_a_ref, idx_ref, out_ref, pre_scr):
    prob_pb = prob_pb_ref[...]

    def pre_store(s_local, row):
        pre_scr[pl.ds(s_local, 1), :] = row

    _sample_half_rows(prob_pb, HALF, idx_ref, pre_store)

    pre = jnp.concatenate([pre_a_ref[...], pre_scr[...]], axis=0)
    m2 = jnp.max(pre, axis=0, keepdims=True)
    lse = m2 + jnp.log(jnp.sum(jnp.exp(pre - m2), axis=0, keepdims=True))
    out_ref[...] = pre - lse


_sample_call_a = pl.pallas_call(
    _sample_a_body,
    out_shape=[
        jax.ShapeDtypeStruct((HALF, B), jnp.int32),
        jax.ShapeDtypeStruct((HALF, B), jnp.float32),
    ],
)

_sample_call_b = pl.pallas_call(
    _sample_b_body,
    out_shape=[
        jax.ShapeDtypeStruct((HALF, B), jnp.int32),
        jax.ShapeDtypeStruct((P, B), jnp.float32),
    ],
    scratch_shapes=[pltpu.VMEM((HALF, B), jnp.float32)],
)

NC = 2
NS = 16
NW = NC * NS
ROWS_HALF = HALF * B
ROWS_W = ROWS_HALF // NW
CH = 128
NCHUNK = ROWS_W // CH
IDX_ROWS_W = ROWS_W // B


@functools.lru_cache(maxsize=None)
def _make_gather_half(half):
    mesh = plsc.VectorSubcoreMesh(
        core_axis_name="c", subcore_axis_name="s", num_cores=NC, num_subcores=NS
    )
    out_base = half * ROWS_HALF

    @functools.partial(
        pl.kernel,
        mesh=mesh,
        out_type=(),
        scratch_types=[
            pltpu.VMEM((NCHUNK, CH), jnp.int32),
            pltpu.VMEM((2, CH, H), jnp.float32),
            pltpu.SemaphoreType.DMA,
            pltpu.SemaphoreType.DMA,
        ],
    )
    def gather_half(idx_hbm, parts_hbm, out_hbm, idx_v, buf_v, sem0, sem1):
        wid = lax.axis_index("s") * NC + lax.axis_index("c")
        pltpu.sync_copy(idx_hbm.at[pl.ds(wid * IDX_ROWS_W, IDX_ROWS_W)], idx_v)
        sems = (sem0, sem1)
        copies = [None, None]
        copies[0] = pltpu.async_copy(parts_hbm.at[idx_v.at[0]], buf_v.at[0], sem0)
        for c in range(NCHUNK):
            nxt = c + 1
            if nxt < NCHUNK:
                copies[nxt % 2] = pltpu.async_copy(
                    parts_hbm.at[idx_v.at[nxt]], buf_v.at[nxt % 2], sems[nxt % 2]
                )
            copies[c % 2].wait()
            pltpu.sync_copy(
                buf_v.at[c % 2],
                out_hbm.at[pl.ds(out_base + wid * ROWS_W + c * CH, CH)],
            )

    return gather_half


def kernel(particles, prob):
    prob_pb = prob.reshape(P, B)
    idx_a, pre_a = _sample_call_a(prob_pb)
    out_ref = jax.new_ref(jnp.zeros((PB, H), jnp.float32))
    _make_gather_half(0)(idx_a, particles, out_ref)
    idx_b, prob_new = _sample_call_b(prob_pb, pre_a)
    _make_gather_half(1)(idx_b, particles, out_ref)
    particles_new = jax.freeze(out_ref)
    return particles_new, prob_new.reshape(P, B, 1)


# speedup vs baseline: 1.0926x; 1.0874x over previous
import functools

import jax
import jax.numpy as jnp
import numpy as np
from jax import lax
from jax.experimental import pallas as pl
from jax.experimental.pallas import tpu as pltpu
from jax.experimental.pallas import tpu_sc as plsc

P = 128
B = 128
H = 256
PB = P * B
ALPHA = 0.5
UNIF_CONST = (1.0 - ALPHA) / P

HALF = P // 2
UNROLL = 4

_TINY = np.float32(1.1754943508222875e-38)
_ONE_MINUS_TINY = np.float32(1.0)


def _threefry2x32_zero(cnt):
    k0 = jnp.uint32(0)
    k1 = jnp.uint32(42)
    ks = [k0, k1, k0 ^ k1 ^ jnp.uint32(0x1BD11BDA)]
    rot_groups = ((13, 15, 26, 6), (17, 29, 16, 24))

    x1 = cnt + ks[1]
    x0 = x1
    first = True
    for i in range(5):
        for r in rot_groups[i % 2]:
            if first:
                first = False
            else:
                x0 = x0 + x1
            x1 = (x1 << jnp.uint32(r)) | (x1 >> jnp.uint32(32 - r))
            x1 = x1 ^ x0
        x0 = x0 + ks[(i + 1) % 3]
        x1 = x1 + ks[(i + 2) % 3] + jnp.uint32(i + 1)
    return x0, x1


def _sample_half_rows(prob_pb, s_base, idx_ref, pre_store):
    l_pb = jnp.log(ALPHA * jnp.exp(prob_pb) + UNIF_CONST)
    d = prob_pb - l_pb

    j_i = lax.broadcasted_iota(jnp.uint32, (P, B), 0)
    b_i = lax.broadcasted_iota(jnp.uint32, (P, B), 1)
    cnt0 = b_i * jnp.uint32(P) + j_i
    jcol = lax.broadcasted_iota(jnp.int32, (P, B), 0)
    bline = lax.broadcasted_iota(jnp.int32, (1, B), 1)

    def one_row(s_local):
        s = s_local + s_base
        cnt = cnt0 + (s * (B * P)).astype(jnp.uint32)
        o0, o1 = _threefry2x32_zero(cnt)
        bits = o0 ^ o1

        float_bits = (bits >> jnp.uint32(9)) | jnp.uint32(0x3F800000)
        floats = lax.bitcast_convert_type(float_bits, jnp.float32)
        floats = floats - jnp.float32(1.0)
        u = jnp.maximum(_TINY, floats * _ONE_MINUS_TINY + _TINY)
        g = -jnp.log(-jnp.log(u))

        vals = g + l_pb
        m = jnp.max(vals, axis=0, keepdims=True)
        cand = jnp.where(vals == m, jcol, jnp.int32(P))
        idx_row = jnp.min(cand, axis=0, keepdims=True)

        idx_ref[pl.ds(s_local, 1), :] = bline + idx_row * B

        mask = idx_row == jcol
        pre_store(
            s_local,
            jnp.sum(jnp.where(mask, d, jnp.float32(0.0)), axis=0, keepdims=True),
        )

    def step(k, _):
        for r in range(UNROLL):
            one_row(UNROLL * k + r)
        return 0

    lax.fori_loop(0, HALF // UNROLL, step, 0)


def _sample_a_body(prob_pb_ref, idx_ref, pre_ref):
    prob_pb = prob_pb_ref[...]

    def pre_store(s_local, row):
        pre_ref[pl.ds(s_local, 1), :] = row

    _sample_half_rows(prob_pb, 0, idx_ref, pre_store)


def _sample_b_body(prob_pb_ref, pre_a_ref, idx_ref, out_ref, pre_scr):
    prob_pb = prob_pb_ref[...]

    def pre_store(s_local, row):
        pre_scr[pl.ds(s_local, 1), :] = row

    _sample_half_rows(prob_pb, HALF, idx_ref, pre_store)

    pre = jnp.concatenate([pre_a_ref[...], pre_scr[...]], axis=0)
    m2 = jnp.max(pre, axis=0, keepdims=True)
    lse = m2 + jnp.log(jnp.sum(jnp.exp(pre - m2), axis=0, keepdims=True))
    out_ref[...] = pre - lse


_sample_call_a = pl.pallas_call(
    _sample_a_body,
    out_shape=[
        jax.ShapeDtypeStruct((HALF, B), jnp.int32),
        jax.ShapeDtypeStruct((HALF, B), jnp.float32),
    ],
)

_sample_call_b = pl.pallas_call(
    _sample_b_body,
    out_shape=[
        jax.ShapeDtypeStruct((HALF, B), jnp.int32),
        jax.ShapeDtypeStruct((P, B), jnp.float32),
    ],
    scratch_shapes=[pltpu.VMEM((HALF, B), jnp.float32)],
)

NC = 2
NS = 16
NW = NC * NS
ROWS_HALF = HALF * B
ROWS_W = ROWS_HALF // NW
CH = 128
NCHUNK = ROWS_W // CH
IDX_ROWS_W = ROWS_W // B


@functools.lru_cache(maxsize=None)
def _make_gather_half(half):
    mesh = plsc.VectorSubcoreMesh(
        core_axis_name="c", subcore_axis_name="s", num_cores=NC, num_subcores=NS
    )
    out_base = half * ROWS_HALF

    @functools.partial(
        pl.kernel,
        mesh=mesh,
        out_type=(),
        scratch_types=[
            pltpu.VMEM((NCHUNK, CH), jnp.int32),
            pltpu.VMEM((2, CH, H), jnp.float32),
            pltpu.SemaphoreType.DMA,
            pltpu.SemaphoreType.DMA,
        ],
    )
    def gather_half(idx_hbm, parts_hbm, out_hbm, idx_v, buf_v, sem0, sem1):
        wid = lax.axis_index("s") * NC + lax.axis_index("c")
        pltpu.sync_copy(idx_hbm.at[pl.ds(wid * IDX_ROWS_W, IDX_ROWS_W)], idx_v)
        sems = (sem0, sem1)
        copies = [None, None]
        copies[0] = pltpu.async_copy(parts_hbm.at[idx_v.at[0]], buf_v.at[0], sem0)
        for c in range(NCHUNK):
            nxt = c + 1
            if nxt < NCHUNK:
                copies[nxt % 2] = pltpu.async_copy(
                    parts_hbm.at[idx_v.at[nxt]], buf_v.at[nxt % 2], sems[nxt % 2]
                )
            copies[c % 2].wait()
            pltpu.sync_copy(
                buf_v.at[c % 2],
                out_hbm.at[pl.ds(out_base + wid * ROWS_W + c * CH, CH)],
            )

    return gather_half


def kernel(particles, prob):
    prob_pb = prob.reshape(P, B)
    idx_a, pre_a = _sample_call_a(prob_pb)
    out_ref = jax.new_ref(lax.empty((PB, H), jnp.float32))
    _make_gather_half(0)(idx_a, particles, out_ref)
    idx_b, prob_new = _sample_call_b(prob_pb, pre_a)
    _make_gather_half(1)(idx_b, particles, out_ref)
    particles_new = jax.freeze(out_ref)
    return particles_new, prob_new.reshape(P, B, 1)


# speedup vs baseline: 1.0928x; 1.0002x over previous
import functools

import jax
import jax.numpy as jnp
import numpy as np
from jax import lax
from jax.experimental import pallas as pl
from jax.experimental.pallas import tpu as pltpu
from jax.experimental.pallas import tpu_sc as plsc

P = 128
B = 128
H = 256
PB = P * B
ALPHA = 0.5
UNIF_CONST = (1.0 - ALPHA) / P

HALF = P // 2
UNROLL = 4

_TINY = np.float32(1.1754943508222875e-38)
_ONE_MINUS_TINY = np.float32(1.0)


def _threefry2x32_zero(cnt):
    k0 = jnp.uint32(0)
    k1 = jnp.uint32(42)
    ks = [k0, k1, k0 ^ k1 ^ jnp.uint32(0x1BD11BDA)]
    rot_groups = ((13, 15, 26, 6), (17, 29, 16, 24))

    x1 = cnt + ks[1]
    x0 = x1
    first = True
    for i in range(5):
        for r in rot_groups[i % 2]:
            if first:
                first = False
            else:
                x0 = x0 + x1
            x1 = (x1 << jnp.uint32(r)) | (x1 >> jnp.uint32(32 - r))
            x1 = x1 ^ x0
        x0 = x0 + ks[(i + 1) % 3]
        x1 = x1 + ks[(i + 2) % 3] + jnp.uint32(i + 1)
    return x0, x1


def _sample_half_rows(prob_pb, s_base, idx_ref, pre_store):
    l_pb = jnp.log(ALPHA * jnp.exp(prob_pb) + UNIF_CONST)
    d = prob_pb - l_pb

    j_i = lax.broadcasted_iota(jnp.uint32, (P, B), 0)
    b_i = lax.broadcasted_iota(jnp.uint32, (P, B), 1)
    cnt0 = b_i * jnp.uint32(P) + j_i
    jcol = lax.broadcasted_iota(jnp.int32, (P, B), 0)
    bline = lax.broadcasted_iota(jnp.int32, (1, B), 1)

    def one_row(s_local):
        s = s_local + s_base
        cnt = cnt0 + (s * (B * P)).astype(jnp.uint32)
        o0, o1 = _threefry2x32_zero(cnt)
        bits = o0 ^ o1

        float_bits = (bits >> jnp.uint32(9)) | jnp.uint32(0x3F800000)
        floats = lax.bitcast_convert_type(float_bits, jnp.float32)
        floats = floats - jnp.float32(1.0)
        u = jnp.maximum(_TINY, floats * _ONE_MINUS_TINY + _TINY)
        g = -jnp.log(-jnp.log(u))

        vals = g + l_pb
        m = jnp.max(vals, axis=0, keepdims=True)
        cand = jnp.where(vals == m, jcol, jnp.int32(P))
        idx_row = jnp.min(cand, axis=0, keepdims=True)

        idx_ref[pl.ds(s_local, 1), :] = bline + idx_row * B

        mask = idx_row == jcol
        pre_store(
            s_local,
            jnp.sum(jnp.where(mask, d, jnp.float32(0.0)), axis=0, keepdims=True),
        )

    def step(k, _):
        for r in range(UNROLL):
            one_row(UNROLL * k + r)
        return 0

    lax.fori_loop(0, HALF // UNROLL, step, 0)


def _sample_a_body(prob_pb_ref, idx_ref, pre_ref):
    prob_pb = prob_pb_ref[...]

    def pre_store(s_local, row):
        pre_ref[pl.ds(s_local, 1), :] = row

    _sample_half_rows(prob_pb, 0, idx_ref, pre_store)


def _sample_b_body(prob_pb_ref, pre_a_ref, idx_ref, out_ref, pre_scr):
    prob_pb = prob_pb_ref[...]

    def pre_store(s_local, row):
        pre_scr[pl.ds(s_local, 1), :] = row

    _sample_half_rows(prob_pb, HALF, idx_ref, pre_store)

    pre = jnp.concatenate([pre_a_ref[...], pre_scr[...]], axis=0)
    m2 = jnp.max(pre, axis=0, keepdims=True)
    lse = m2 + jnp.log(jnp.sum(jnp.exp(pre - m2), axis=0, keepdims=True))
    out_ref[...] = pre - lse


_sample_call_a = pl.pallas_call(
    _sample_a_body,
    out_shape=[
        jax.ShapeDtypeStruct((HALF, B), jnp.int32),
        jax.ShapeDtypeStruct((HALF, B), jnp.float32),
    ],
)

_sample_call_b = pl.pallas_call(
    _sample_b_body,
    out_shape=[
        jax.ShapeDtypeStruct((HALF, B), jnp.int32),
        jax.ShapeDtypeStruct((P, B), jnp.float32),
    ],
    scratch_shapes=[pltpu.VMEM((HALF, B), jnp.float32)],
)

NC = 2
NS = 16
NW = NC * NS
ROWS_HALF = HALF * B
ROWS_W = ROWS_HALF // NW
CH = 128
NCHUNK = ROWS_W // CH
IDX_ROWS_W = ROWS_W // B


@functools.lru_cache(maxsize=None)
def _make_gather_half(half):
    mesh = plsc.VectorSubcoreMesh(
        core_axis_name="c", subcore_axis_name="s", num_cores=NC, num_subcores=NS
    )
    out_base = half * ROWS_HALF

    @functools.partial(
        pl.kernel,
        mesh=mesh,
        out_type=(),
        scratch_types=[
            pltpu.VMEM((NCHUNK, CH), jnp.int32),
            pltpu.VMEM((NCHUNK, CH, H), jnp.float32),
            pltpu.SemaphoreType.DMA,
            pltpu.SemaphoreType.DMA,
            pltpu.SemaphoreType.DMA,
            pltpu.SemaphoreType.DMA,
        ],
    )
    def gather_half(idx_hbm, parts_hbm, out_hbm, idx_v, buf_v, gs0, gs1, ws0, ws1):
        wid = lax.axis_index("s") * NC + lax.axis_index("c")
        pltpu.sync_copy(idx_hbm.at[pl.ds(wid * IDX_ROWS_W, IDX_ROWS_W)], idx_v)
        gsems = (gs0, gs1)
        wsems = (ws0, ws1)
        gathers = [
            pltpu.async_copy(parts_hbm.at[idx_v.at[c]], buf_v.at[c], gsems[c])
            for c in range(NCHUNK)
        ]
        scatters = []
        for c in range(NCHUNK):
            gathers[c].wait()
            scatters.append(
                pltpu.async_copy(
                    buf_v.at[c],
                    out_hbm.at[pl.ds(out_base + wid * ROWS_W + c * CH, CH)],
                    wsems[c],
                )
            )
        for s in scatters:
            s.wait()

    return gather_half


def kernel(particles, prob):
    prob_pb = prob.reshape(P, B)
    idx_a, pre_a = _sample_call_a(prob_pb)
    out_ref = jax.new_ref(lax.empty((PB, H), jnp.float32))
    _make_gather_half(0)(idx_a, particles, out_ref)
    idx_b, prob_new = _sample_call_b(prob_pb, pre_a)
    _make_gather_half(1)(idx_b, particles, out_ref)
    particles_new = jax.freeze(out_ref)
    return particles_new, prob_new.reshape(P, B, 1)
